# baseline (speedup 1.0000x reference)
import functools

import jax
import jax.numpy as jnp
from jax import lax
from jax.experimental import pallas as pl
from jax.experimental.pallas import tpu as pltpu
from jax.experimental.pallas import tpu_sc as plsc

E_BLOCK = 3200
N_BLOCK = 1000

SC_CORES = 2
SC_SUBCORES = 16
SC_TILES = SC_CORES * SC_SUBCORES
EDGE_CHUNK = 80
ACC_N = 10240


def _t1_body(ef_ref, wea_ref, bea_ref, wm1_ref, bm1_ref,
             ei0_ref, ei1_ref, e1_ref):
    ef = ef_ref[...]
    ei = jnp.dot(ef, wea_ref[...],
                 preferred_element_type=jnp.float32) + bea_ref[...]
    h2 = ei.shape[1] // 2
    ei0_ref[...] = ei[:, :h2]
    ei1_ref[...] = ei[:, h2:]
    e1_ref[...] = jnp.maximum(
        jnp.dot(ef, wm1_ref[...], preferred_element_type=jnp.float32)
        + bm1_ref[...], 0.0)


def _t1(edge_feats, W_ea, b_ea, W_m1, b_m1):
    E, D = edge_feats.shape
    H = W_ea.shape[1]
    Hh = H // 2
    grid = (E // E_BLOCK,)
    return pl.pallas_call(
        _t1_body,
        grid=grid,
        in_specs=[
            pl.BlockSpec((E_BLOCK, D), lambda i: (i, 0)),
            pl.BlockSpec((D, H), lambda i: (0, 0)),
            pl.BlockSpec((1, H), lambda i: (0, 0)),
            pl.BlockSpec((D, H), lambda i: (0, 0)),
            pl.BlockSpec((1, H), lambda i: (0, 0)),
        ],
        out_specs=[
            pl.BlockSpec((E_BLOCK, Hh), lambda i: (i, 0)),
            pl.BlockSpec((E_BLOCK, Hh), lambda i: (i, 0)),
            pl.BlockSpec((E_BLOCK, H), lambda i: (i, 0)),
        ],
        out_shape=[
            jax.ShapeDtypeStruct((E, Hh), jnp.float32),
            jax.ShapeDtypeStruct((E, Hh), jnp.float32),
            jax.ShapeDtypeStruct((E, H), jnp.float32),
        ],
    )(edge_feats, W_ea, b_ea.reshape(1, H), W_m1, b_m1.reshape(1, H))


def _t2a_body(sr0_ref, sr1_ref, sc0_ref, sc1_ref, cr_ref, cc_ref,
              nf_ref, wg1_ref,
              agg_ref, dinv_ref, xw1_ref, y1p_ref):
    sum_row = jnp.concatenate([sr0_ref[...], sr1_ref[...]], axis=1)
    sum_col = jnp.concatenate([sc0_ref[...], sc1_ref[...]], axis=1)
    cnt_row = cr_ref[...]
    cnt_col = cc_ref[...]
    raw_row = sum_row / jnp.maximum(cnt_row, 1.0)
    raw_col = sum_col / jnp.maximum(cnt_col, 1.0)
    agg_ref[...] = jax.nn.sigmoid(raw_row * raw_col)
    dinv = jax.lax.rsqrt(cnt_col + 1.0)
    dinv_ref[...] = dinv
    xw1 = jnp.dot(nf_ref[...], wg1_ref[...], preferred_element_type=jnp.float32)
    xw1_ref[...] = xw1
    y1 = xw1 * dinv
    y1p_ref[...] = jnp.concatenate(
        [y1, jnp.zeros_like(y1)], axis=1)


def _t2a(sr0, sr1, sc0, sc1, cr, cc, node_feats, W_g1):
    N, Hh = sr0.shape
    H = 2 * Hh
    D = node_feats.shape[1]
    NB = N_BLOCK
    rb = lambda w: pl.BlockSpec((NB, w), lambda i: (i, 0))
    full = lambda a, b: pl.BlockSpec((a, b), lambda i: (0, 0))
    return pl.pallas_call(
        _t2a_body,
        grid=(N // NB,),
        in_specs=[rb(Hh), rb(Hh), rb(Hh), rb(Hh), rb(1), rb(1),
                  rb(D), full(D, H)],
        out_specs=[rb(H), rb(1), rb(H), rb(2 * H)],
        out_shape=[
            jax.ShapeDtypeStruct((N, H), jnp.float32),
            jax.ShapeDtypeStruct((N, 1), jnp.float32),
            jax.ShapeDtypeStruct((N, H), jnp.float32),
            jax.ShapeDtypeStruct((N, 2 * H), jnp.float32),
        ],
    )(sr0, sr1, sc0, sc1, cr, cc, node_feats, W_g1)


def _t2b_body(s1a_ref, s1b_ref, dinv_ref, xw1_ref, agg_ref, wg2_ref, bg1_ref,
              xw2_ref, y2d_ref):
    dinv = dinv_ref[...]
    H = xw1_ref.shape[1]
    s1 = (s1a_ref[...] + s1b_ref[...])[:, :H]
    out1 = s1 * dinv + xw1_ref[...] * (dinv * dinv) + bg1_ref[...]
    x1 = jnp.maximum(out1, 0.0) * agg_ref[...]
    xw2 = jnp.dot(x1, wg2_ref[...], preferred_element_type=jnp.float32)
    xw2_ref[...] = xw2
    y2d_ref[...] = xw2 * dinv


def _t2b(S1a, S1b, dinv, xw1, agg, W_g2, b_g1):
    N, H = xw1.shape
    D = W_g2.shape[1]
    NB = N_BLOCK
    rb = lambda w: pl.BlockSpec((NB, w), lambda i: (i, 0))
    full = lambda a, b: pl.BlockSpec((a, b), lambda i: (0, 0))
    return pl.pallas_call(
        _t2b_body,
        grid=(N // NB,),
        in_specs=[rb(2 * H), rb(2 * H), rb(1), rb(H), rb(H),
                  full(H, D), full(1, H)],
        out_specs=[rb(D), rb(D)],
        out_shape=[
            jax.ShapeDtypeStruct((N, D), jnp.float32),
            jax.ShapeDtypeStruct((N, D), jnp.float32),
        ],
    )(S1a, S1b, dinv, xw1, agg, W_g2, b_g1.reshape(1, H))


def _t2c_body(s2a_ref, s2b_ref, dinv_ref, xw2_ref,
              wna_ref, bna_ref, wnra_ref, wnrb_ref, bnr_ref, bg2_ref,
              x_ref, ab_ref):
    dinv = dinv_ref[...]
    s2 = s2a_ref[...] + s2b_ref[...]
    x = jnp.maximum(s2 * dinv + xw2_ref[...] * (dinv * dinv) + bg2_ref[...],
                    0.0)
    x_ref[...] = x
    node_ind = jnp.maximum(
        jnp.dot(x, wna_ref[...], preferred_element_type=jnp.float32)
        + bna_ref[...], 0.0)
    a = jnp.dot(node_ind, wnra_ref[...],
                preferred_element_type=jnp.float32) + bnr_ref[...]
    b = jnp.dot(node_ind, wnrb_ref[...],
                preferred_element_type=jnp.float32)
    ab_ref[...] = jnp.concatenate([a, b], axis=1)


def _t2c(S2a, S2b, dinv, xw2, W_na, b_na, W_nrA, W_nrB, b_nr, b_g2):
    N, D = xw2.shape
    H = W_na.shape[1]
    NB = N_BLOCK
    rb = lambda w: pl.BlockSpec((NB, w), lambda i: (i, 0))
    full = lambda a, b: pl.BlockSpec((a, b), lambda i: (0, 0))
    return pl.pallas_call(
        _t2c_body,
        grid=(N // NB,),
        in_specs=[rb(D), rb(D),
                  rb(1), rb(D), full(D, H), full(1, H),
                  full(H, H), full(H, H), full(1, H), full(1, D)],
        out_specs=[rb(D), rb(2 * H)],
        out_shape=[
            jax.ShapeDtypeStruct((N, D), jnp.float32),
            jax.ShapeDtypeStruct((N, 2 * H), jnp.float32),
        ],
    )(S2a, S2b, dinv, xw2, W_na, b_na.reshape(1, H),
      W_nrA, W_nrB, b_nr.reshape(1, H), b_g2.reshape(1, D))


def _t3_body(e1_ref, ga_ref, gb_ref, wm2_ref, bm2_ref, e_ref):
    H = e1_ref.shape[1]
    p = ga_ref[...][:, :H] + gb_ref[...][:, H:]
    g = e1_ref[...] * jax.nn.sigmoid(p)
    e_ref[...] = jnp.maximum(
        jnp.dot(g, wm2_ref[...], preferred_element_type=jnp.float32)
        + bm2_ref[...], 0.0)


def _t3(e1, GAx, GBx, W_m2, b_m2):
    E, H = e1.shape
    D = W_m2.shape[1]
    grid = (E // E_BLOCK,)
    return pl.pallas_call(
        _t3_body,
        grid=grid,
        in_specs=[
            pl.BlockSpec((E_BLOCK, H), lambda i: (i, 0)),
            pl.BlockSpec((E_BLOCK, 2 * H), lambda i: (i, 0)),
            pl.BlockSpec((E_BLOCK, 2 * H), lambda i: (i, 0)),
            pl.BlockSpec((H, D), lambda i: (0, 0)),
            pl.BlockSpec((1, D), lambda i: (0, 0)),
        ],
        out_specs=pl.BlockSpec((E_BLOCK, D), lambda i: (i, 0)),
        out_shape=jax.ShapeDtypeStruct((E, D), jnp.float32),
    )(e1, GAx, GBx, W_m2, b_m2.reshape(1, D))


def _sc_mesh():
    return plsc.VectorSubcoreMesh(core_axis_name="c", subcore_axis_name="s")


def _k1_scatter_stats(ei0, ei1, src3d, dst3d):
    E, Hh = ei0.shape
    _, nchunk, C = src3d.shape
    per_tile = nchunk * C
    wb = ACC_N // SC_SUBCORES
    zeros_h = jnp.zeros((wb, Hh), jnp.float32)

    @functools.partial(
        pl.kernel,
        out_type=[
            jax.ShapeDtypeStruct((SC_CORES, ACC_N, Hh), jnp.float32),
            jax.ShapeDtypeStruct((SC_CORES, ACC_N, Hh), jnp.float32),
        ],
        mesh=_sc_mesh(),
        scratch_types=[
            pltpu.VMEM_SHARED((ACC_N, Hh), jnp.float32),
            pltpu.VMEM_SHARED((ACC_N, Hh), jnp.float32),
            pltpu.VMEM((nchunk, C), jnp.int32),
            pltpu.VMEM((nchunk, C), jnp.int32),
            pltpu.VMEM((C, Hh), jnp.float32),
        ],
    )
    def k1(ei0_hbm, ei1_hbm, src_hbm, dst_hbm, z_hbm,
           sr_out, sc_out,
           acc_sr, acc_sc, src_v, dst_v, vals_v):
        cid = lax.axis_index("c")
        sid = lax.axis_index("s")
        base = sid * wb
        ebase = sid * per_tile
        pltpu.sync_copy(z_hbm, acc_sr.at[pl.ds(base, wb)])
        pltpu.sync_copy(z_hbm, acc_sc.at[pl.ds(base, wb)])
        pltpu.sync_copy(src_hbm.at[sid], src_v)
        pltpu.sync_copy(dst_hbm.at[sid], dst_v)
        plsc.subcore_barrier()

        @pl.loop(0, nchunk)
        def _(j):
            @pl.when(cid == 0)
            def _():
                pltpu.sync_copy(ei0_hbm.at[pl.ds(ebase + j * C, C)], vals_v)

            @pl.when(cid == 1)
            def _():
                pltpu.sync_copy(ei1_hbm.at[pl.ds(ebase + j * C, C)], vals_v)

            pltpu.sync_copy(vals_v, acc_sr.at[src_v.at[j]], add=True)
            pltpu.sync_copy(vals_v, acc_sc.at[dst_v.at[j]], add=True)

        plsc.subcore_barrier()
        pltpu.sync_copy(acc_sr.at[pl.ds(base, wb)],
                        sr_out.at[cid, pl.ds(base, wb)])
        pltpu.sync_copy(acc_sc.at[pl.ds(base, wb)],
                        sc_out.at[cid, pl.ds(base, wb)])

    return k1(ei0, ei1, src3d, dst3d, zeros_h)


def _k1b_counts(src3d, dst3d):
    _, nchunk, C = src3d.shape
    wb = ACC_N // SC_SUBCORES
    ones_h = jnp.ones((C, 16), jnp.float32)
    zeros_c = jnp.zeros((wb, 16), jnp.float32)

    @functools.partial(
        pl.kernel,
        out_type=[
            jax.ShapeDtypeStruct((ACC_N, 16), jnp.float32),
            jax.ShapeDtypeStruct((ACC_N, 16), jnp.float32),
        ],
        mesh=_sc_mesh(),
        scratch_types=[
            pltpu.VMEM_SHARED((ACC_N, 16), jnp.float32),
            pltpu.VMEM((C, 16), jnp.float32),
            pltpu.VMEM((nchunk, C), jnp.int32),
        ],
    )
    def k1b(src_hbm, dst_hbm, ones_hbm, zc_hbm, cr_out, cc_out,
            acc_cnt, ones_v, idx_v):
        cid = lax.axis_index("c")
        sid = lax.axis_index("s")
        base = sid * wb
        pltpu.sync_copy(zc_hbm, acc_cnt.at[pl.ds(base, wb)])
        pltpu.sync_copy(ones_hbm, ones_v)

        @pl.when(cid == 0)
        def _():
            pltpu.sync_copy(src_hbm.at[sid], idx_v)

        @pl.when(cid == 1)
        def _():
            pltpu.sync_copy(dst_hbm.at[sid], idx_v)

        plsc.subcore_barrier()

        @pl.loop(0, nchunk)
        def _(j):
            pltpu.sync_copy(ones_v, acc_cnt.at[idx_v.at[j]], add=True)

        plsc.subcore_barrier()

        @pl.when(cid == 0)
        def _():
            pltpu.sync_copy(acc_cnt.at[pl.ds(base, wb)],
                            cr_out.at[pl.ds(base, wb)])

        @pl.when(cid == 1)
        def _():
            pltpu.sync_copy(acc_cnt.at[pl.ds(base, wb)],
                            cc_out.at[pl.ds(base, wb)])

    return k1b(src3d, dst3d, ones_h, zeros_c)


def _k_msg(table, src3d32, dst3d32, passes):
    Ntab, F = table.shape
    Fh = F // 2
    _, nchunk, C = src3d32.shape
    per_tile = nchunk * C
    wb = ACC_N // SC_SUBCORES
    zeros_h = jnp.zeros((wb, Fh), jnp.float32)

    @functools.partial(
        pl.kernel,
        out_type=jax.ShapeDtypeStruct((SC_CORES, passes, ACC_N, Fh),
                                      jnp.float32),
        mesh=_sc_mesh(),
        scratch_types=[
            pltpu.VMEM_SHARED((ACC_N, Fh), jnp.float32),
            pltpu.VMEM((C, F), jnp.float32),
            pltpu.VMEM((C, Fh), jnp.float32),
            pltpu.VMEM((nchunk, C), jnp.int32),
            pltpu.VMEM((nchunk, C), jnp.int32),
        ],
    )
    def kmsg(tab_hbm, src_hbm, dst_hbm, z_hbm, s_out,
             acc, vals_v, half_v, src_v, dst_v):
        cid = lax.axis_index("c")
        sid = lax.axis_index("s")
        wid = cid * SC_SUBCORES + sid
        base = sid * wb
        pltpu.sync_copy(src_hbm.at[wid], src_v)
        pltpu.sync_copy(dst_hbm.at[wid], dst_v)
        for p in range(passes):
            pltpu.sync_copy(z_hbm, acc.at[pl.ds(base, wb)])
            plsc.subcore_barrier()

            @pl.loop(0, nchunk)
            def _(j):
                pltpu.sync_copy(tab_hbm.at[src_v.at[j]], vals_v)

                @pl.loop(0, C)
                def _(r):
                    for cc0 in range(Fh // 16):
                        half_v[r, pl.ds(cc0 * 16, 16)] = vals_v[
                            r, pl.ds(p * Fh + cc0 * 16, 16)]

                pltpu.sync_copy(half_v, acc.at[dst_v.at[j]], add=True)

            plsc.subcore_barrier()
            pltpu.sync_copy(acc.at[pl.ds(base, wb)],
                            s_out.at[cid, p, pl.ds(base, wb)])
            plsc.subcore_barrier()

    return kmsg(table, src3d32, dst3d32, zeros_h)


def _k4_pair_gather(AB, src3d32, dst3d32):
    Ntab, F = AB.shape
    _, nchunk, C = src3d32.shape
    E = SC_TILES * nchunk * C
    per_tile = nchunk * C

    @functools.partial(
        pl.kernel,
        out_type=[
            jax.ShapeDtypeStruct((E, F), jnp.float32),
            jax.ShapeDtypeStruct((E, F), jnp.float32),
        ],
        mesh=_sc_mesh(),
        scratch_types=[
            pltpu.VMEM((nchunk, C), jnp.int32),
            pltpu.VMEM((nchunk, C), jnp.int32),
            pltpu.VMEM((C, F), jnp.float32),
            pltpu.VMEM((C, F), jnp.float32),
            pltpu.VMEM((C, F), jnp.float32),
            pltpu.VMEM((C, F), jnp.float32),
        ],
    )
    def k4(ab_hbm, src_hbm, dst_hbm, ga_out, gb_out,
           src_v, dst_v, buf_a0, buf_b0, buf_a1, buf_b1):
        cid = lax.axis_index("c")
        sid = lax.axis_index("s")
        wid = cid * SC_SUBCORES + sid
        ebase = wid * per_tile
        pltpu.sync_copy(src_hbm.at[wid], src_v)
        pltpu.sync_copy(dst_hbm.at[wid], dst_v)

        @pl.loop(0, nchunk // 2)
        def _(jj):
            for k, (ba, bb) in enumerate(((buf_a0, buf_b0),
                                          (buf_a1, buf_b1))):
                j = jj * 2 + k
                pltpu.sync_copy(ab_hbm.at[src_v.at[j]], ba)
                pltpu.sync_copy(ba, ga_out.at[pl.ds(ebase + j * C, C)])
                pltpu.sync_copy(ab_hbm.at[dst_v.at[j]], bb)
                pltpu.sync_copy(bb, gb_out.at[pl.ds(ebase + j * C, C)])

        if nchunk % 2:
            j = nchunk - 1
            pltpu.sync_copy(ab_hbm.at[src_v.at[j]], buf_a0)
            pltpu.sync_copy(buf_a0, ga_out.at[pl.ds(ebase + j * C, C)])
            pltpu.sync_copy(ab_hbm.at[dst_v.at[j]], buf_b0)
            pltpu.sync_copy(buf_b0, gb_out.at[pl.ds(ebase + j * C, C)])

    return k4(AB, src3d32, dst3d32)


def kernel(node_feats, edge_feats, edge_index, W_g1, b_g1, W_g2, b_g2,
           W_ea, b_ea, W_na, b_na, W_nr, b_nr, W_m1, b_m1, W_m2, b_m2):
    src = edge_index[0]
    dst = edge_index[1]
    N, D = node_feats.shape
    H = D // 2
    E = src.shape[0]
    C = EDGE_CHUNK
    nchunk16 = E // (SC_SUBCORES * C)
    nchunk32 = E // (SC_TILES * C)
    src3d = src.reshape(SC_SUBCORES, nchunk16, C)
    dst3d = dst.reshape(SC_SUBCORES, nchunk16, C)
    src3d32 = src.reshape(SC_TILES, nchunk32, C)
    dst3d32 = dst.reshape(SC_TILES, nchunk32, C)

    ei0, ei1, e1 = _t1(edge_feats, W_ea, b_ea, W_m1, b_m1)

    srp, scp = _k1_scatter_stats(ei0, ei1, src3d, dst3d)
    crp, ccp = _k1b_counts(src3d, dst3d)
    agg, dinv, xw1, y1p = _t2a(srp[0, :N], srp[1, :N],
                               scp[0, :N], scp[1, :N],
                               crp[:N, :1], ccp[:N, :1],
                               node_feats, W_g1)

    S1f = jnp.zeros_like(y1p).at[dst].add(y1p[src])
    xw2, y2d = _t2b(S1f, jnp.zeros_like(S1f), dinv, xw1, agg, W_g2, b_g1)

    S2p = _k_msg(y2d, src3d32, dst3d32, 2)
    x, AB = _t2c(jnp.concatenate([S2p[0, 0, :N], S2p[0, 1, :N]], axis=1),
                 jnp.concatenate([S2p[1, 0, :N], S2p[1, 1, :N]], axis=1),
                 dinv, xw2, W_na, b_na,
                 W_nr[:H], W_nr[H:], b_nr, b_g2)

    GAx, GBx = _k4_pair_gather(AB, src3d32, dst3d32)
    e = _t3(e1, GAx, GBx, W_m2, b_m2)
    return (x, e)
